# baseline (speedup 1.0000x reference)
import functools

import jax
import jax.numpy as jnp
import numpy as np
from jax import lax
from jax.experimental import pallas as pl


def _colsum(v):
    r = v.shape[0]
    while r > 8:
        half = r // 2
        v = v[:half] + v[half:]
        r = half
    return jnp.sum(v, axis=0, keepdims=True)


def _strip_body(x_ref, o_ref, *, target, lw, hw):
    group = x_ref.shape[0]
    n2 = x_ref[0] * x_ref[0]
    for r in range(1, group):
        plane = x_ref[r]
        n2 = n2 + plane * plane
    qblk = n2.shape[1]

    bits = lax.bitcast_convert_type(n2, jnp.int32)

    tbits = jnp.zeros((1, qblk), jnp.int32)
    for b in range(30, -1, -1):
        cand = tbits | jnp.int32(1 << b)
        cnt = _colsum((bits >= cand).astype(jnp.int32))
        tbits = jnp.where(cnt >= target, cand, tbits)
    cnt_t = _colsum((bits >= tbits).astype(jnp.int32))
    v_hi = lax.bitcast_convert_type(tbits, jnp.float32)
    below = jnp.where(bits < tbits, bits, 0)
    r = below.shape[0]
    while r > 8:
        below = jnp.maximum(below[:r // 2], below[r // 2:])
        r //= 2
    below = jnp.max(below, axis=0, keepdims=True)
    v_lo = jnp.where(cnt_t >= target + 1, v_hi,
                     lax.bitcast_convert_type(below, jnp.float32))
    thres = jnp.sqrt(v_lo) * lw + jnp.sqrt(v_hi) * hw

    keep = (jnp.sqrt(n2) >= thres).astype(jnp.float32)
    for r in range(group):
        o_ref[r] = x_ref[r] * keep


@jax.jit
def kernel(x):
    p, q, k1, k2 = x.shape
    group = k1 * k2
    qblk = 128
    grid = q // qblk
    xt = x.transpose(2, 3, 0, 1).reshape(group, p, q)

    qs = np.float32(0.9) * np.float32(p - 1)
    low = np.floor(qs)
    hw = np.float32(qs - low)
    lw = np.float32(np.float32(1.0) - hw)
    target = int(p - np.ceil(qs))

    body = functools.partial(_strip_body, target=target, lw=lw, hw=hw)
    out = pl.pallas_call(
        body,
        grid=(grid,),
        in_specs=[pl.BlockSpec((group, p, qblk), lambda i: (0, 0, i))],
        out_specs=pl.BlockSpec((group, p, qblk), lambda i: (0, 0, i)),
        out_shape=jax.ShapeDtypeStruct((group, p, q), jnp.float32),
    )(xt)
    return out.reshape(k1, k2, p, q).transpose(2, 3, 0, 1)


# speedup vs baseline: 1.5744x; 1.5744x over previous
import functools

import jax
import jax.numpy as jnp
import numpy as np
from jax import lax
from jax.experimental import pallas as pl


def _colsum(v):
    chunk = max(v.shape[0] // 16, 8)
    parts = [jnp.sum(v[i:i + chunk], axis=0, keepdims=True)
             for i in range(0, v.shape[0], chunk)]
    while len(parts) > 1:
        parts = [parts[j] + parts[j + 1] if j + 1 < len(parts) else parts[j]
                 for j in range(0, len(parts), 2)]
    return parts[0]


def _strip_body(x_ref, o_ref, *, target, lw, hw):
    group = x_ref.shape[0]
    n2 = x_ref[0] * x_ref[0]
    for r in range(1, group):
        plane = x_ref[r]
        n2 = n2 + plane * plane
    qblk = n2.shape[1]

    bits = lax.bitcast_convert_type(n2, jnp.int32)

    tbits = jnp.zeros((1, qblk), jnp.int32)
    for b in range(30, -1, -1):
        cand = tbits | jnp.int32(1 << b)
        cnt = _colsum((bits >= cand).astype(jnp.int32))
        tbits = jnp.where(cnt >= target, cand, tbits)
    cnt_t = _colsum((bits >= tbits).astype(jnp.int32))
    v_hi = lax.bitcast_convert_type(tbits, jnp.float32)
    below = jnp.where(bits < tbits, bits, 0)
    r = below.shape[0]
    while r > 8:
        below = jnp.maximum(below[:r // 2], below[r // 2:])
        r //= 2
    below = jnp.max(below, axis=0, keepdims=True)
    v_lo = jnp.where(cnt_t >= target + 1, v_hi,
                     lax.bitcast_convert_type(below, jnp.float32))
    thres = jnp.sqrt(v_lo) * lw + jnp.sqrt(v_hi) * hw

    keep = (jnp.sqrt(n2) >= thres).astype(jnp.float32)
    for r in range(group):
        o_ref[r] = x_ref[r] * keep


@jax.jit
def kernel(x):
    p, q, k1, k2 = x.shape
    group = k1 * k2
    qblk = 128
    grid = q // qblk
    xt = x.transpose(2, 3, 0, 1).reshape(group, p, q)

    qs = np.float32(0.9) * np.float32(p - 1)
    low = np.floor(qs)
    hw = np.float32(qs - low)
    lw = np.float32(np.float32(1.0) - hw)
    target = int(p - np.ceil(qs))

    body = functools.partial(_strip_body, target=target, lw=lw, hw=hw)
    out = pl.pallas_call(
        body,
        grid=(grid,),
        in_specs=[pl.BlockSpec((group, p, qblk), lambda i: (0, 0, i))],
        out_specs=pl.BlockSpec((group, p, qblk), lambda i: (0, 0, i)),
        out_shape=jax.ShapeDtypeStruct((group, p, q), jnp.float32),
    )(xt)
    return out.reshape(k1, k2, p, q).transpose(2, 3, 0, 1)


# speedup vs baseline: 1.7011x; 1.0805x over previous
import functools

import jax
import jax.numpy as jnp
import numpy as np
from jax import lax
from jax.experimental import pallas as pl


def _colsum(v):
    chunk = max(v.shape[0] // 16, 8)
    parts = [jnp.sum(v[i:i + chunk], axis=0, keepdims=True)
             for i in range(0, v.shape[0], chunk)]
    while len(parts) > 1:
        parts = [parts[j] + parts[j + 1] if j + 1 < len(parts) else parts[j]
                 for j in range(0, len(parts), 2)]
    return parts[0]


def _colsum16(v):
    r = v.shape[0]
    while r > 16:
        half = r // 2
        v = v[:half] + v[half:]
        r = half
    return _colsum(v.astype(jnp.int32))


def _strip_body(x_ref, o_ref, *, target, lw, hw):
    group = x_ref.shape[0]
    n2 = x_ref[0] * x_ref[0]
    for r in range(1, group):
        plane = x_ref[r]
        n2 = n2 + plane * plane
    qblk = n2.shape[1]

    bits = lax.bitcast_convert_type(n2, jnp.int32)

    bh = lax.shift_right_logical(bits, 16).astype(jnp.int16)
    t16 = jnp.zeros((1, qblk), jnp.int32)
    for b in range(14, -1, -1):
        cand = t16 | jnp.int32(1 << b)
        cnt = _colsum16((bh >= cand.astype(jnp.int16)).astype(jnp.int16))
        t16 = jnp.where(cnt >= target, cand, t16)
    t16_16 = t16.astype(jnp.int16)
    cnt_gt = _colsum16((bh > t16_16).astype(jnp.int16))
    eqm = (bh == t16_16).astype(jnp.int16)
    target2 = jnp.int32(target) - cnt_gt
    lo_s = ((bits & jnp.int32(0xFFFF)) ^ jnp.int32(0x8000)).astype(jnp.int16)
    t2 = jnp.zeros((1, qblk), jnp.int32)
    for b in range(15, -1, -1):
        cand = t2 | jnp.int32(1 << b)
        cand16 = (cand ^ jnp.int32(0x8000)).astype(jnp.int16)
        contrib = jnp.where(lo_s >= cand16, eqm, jnp.int16(0))
        cnt = _colsum16(contrib)
        t2 = jnp.where(cnt >= target2, cand, t2)
    tbits = lax.shift_left(t16, 16) | t2
    t2_16 = (t2 ^ jnp.int32(0x8000)).astype(jnp.int16)
    cnt_t = cnt_gt + _colsum16(jnp.where(lo_s >= t2_16, eqm, jnp.int16(0)))
    v_hi = lax.bitcast_convert_type(tbits, jnp.float32)
    below = jnp.where(bits < tbits, bits, 0)
    r = below.shape[0]
    while r > 8:
        below = jnp.maximum(below[:r // 2], below[r // 2:])
        r //= 2
    below = jnp.max(below, axis=0, keepdims=True)
    v_lo = jnp.where(cnt_t >= target + 1, v_hi,
                     lax.bitcast_convert_type(below, jnp.float32))
    thres = jnp.sqrt(v_lo) * lw + jnp.sqrt(v_hi) * hw

    keep = (jnp.sqrt(n2) >= thres).astype(jnp.float32)
    for r in range(group):
        o_ref[r] = x_ref[r] * keep


@jax.jit
def kernel(x):
    p, q, k1, k2 = x.shape
    group = k1 * k2
    qblk = 128
    grid = q // qblk
    xt = x.transpose(2, 3, 0, 1).reshape(group, p, q)

    qs = np.float32(0.9) * np.float32(p - 1)
    low = np.floor(qs)
    hw = np.float32(qs - low)
    lw = np.float32(np.float32(1.0) - hw)
    target = int(p - np.ceil(qs))

    body = functools.partial(_strip_body, target=target, lw=lw, hw=hw)
    out = pl.pallas_call(
        body,
        grid=(grid,),
        in_specs=[pl.BlockSpec((group, p, qblk), lambda i: (0, 0, i))],
        out_specs=pl.BlockSpec((group, p, qblk), lambda i: (0, 0, i)),
        out_shape=jax.ShapeDtypeStruct((group, p, q), jnp.float32),
    )(xt)
    return out.reshape(k1, k2, p, q).transpose(2, 3, 0, 1)


# speedup vs baseline: 1.8558x; 1.0909x over previous
import functools

import jax
import jax.numpy as jnp
import numpy as np
from jax import lax
from jax.experimental import pallas as pl

_SLAB = 64


def _colsum(v):
    chunk = max(v.shape[0] // 16, 8)
    parts = [jnp.sum(v[i:i + chunk], axis=0, keepdims=True)
             for i in range(0, v.shape[0], chunk)]
    while len(parts) > 1:
        parts = [parts[j] + parts[j + 1] if j + 1 < len(parts) else parts[j]
                 for j in range(0, len(parts), 2)]
    return parts[0]


def _tree(parts, op):
    while len(parts) > 1:
        parts = [op(parts[j], parts[j + 1]) if j + 1 < len(parts) else parts[j]
                 for j in range(0, len(parts), 2)]
    return parts[0]


def _fold16(c):
    c = c[:32] + c[32:]
    return c[:16] + c[16:]


def _count16(slabs, pred):
    parts = [_fold16(pred(sl)) for sl in slabs]
    total = _tree(parts, lambda a, b: a + b)
    return _colsum(total.astype(jnp.int32))


def _strip_body(x_ref, o_ref, *, target, lw, hw):
    group, rows, qblk = x_ref.shape
    nslab = rows // _SLAB

    n2s, bhs, los = [], [], []
    for s in range(nslab):
        sl = pl.ds(s * _SLAB, _SLAB)
        acc = x_ref[0, sl] * x_ref[0, sl]
        for r in range(1, group):
            plane = x_ref[r, sl]
            acc = acc + plane * plane
        n2s.append(acc)
        bits = lax.bitcast_convert_type(acc, jnp.int32)
        bhs.append(lax.shift_right_logical(bits, 16).astype(jnp.int16))
        los.append(((bits & jnp.int32(0xFFFF))
                    ^ jnp.int32(0x8000)).astype(jnp.int16))

    t16 = jnp.zeros((1, qblk), jnp.int32)
    for b in range(14, -1, -1):
        cand = t16 | jnp.int32(1 << b)
        cand16 = cand.astype(jnp.int16)
        cnt = _count16(bhs, lambda sl: (sl >= cand16).astype(jnp.int16))
        t16 = jnp.where(cnt >= target, cand, t16)
    t16_16 = t16.astype(jnp.int16)
    cnt_gt = _count16(bhs, lambda sl: (sl > t16_16).astype(jnp.int16))
    target2 = jnp.int32(target) - cnt_gt

    lo2s = [jnp.where(bh == t16_16, lo, jnp.int16(-0x8000))
            for bh, lo in zip(bhs, los)]

    t2 = jnp.zeros((1, qblk), jnp.int32)
    for b in range(15, -1, -1):
        cand = t2 | jnp.int32(1 << b)
        cand16 = (cand ^ jnp.int32(0x8000)).astype(jnp.int16)
        cnt = _count16(lo2s, lambda sl: (sl >= cand16).astype(jnp.int16))
        t2 = jnp.where(cnt >= target2, cand, t2)
    tbits = lax.shift_left(t16, 16) | t2

    cnts, belows = [], []
    for s in range(nslab):
        bits = lax.bitcast_convert_type(n2s[s], jnp.int32)
        ge = bits >= tbits
        c = jnp.where(ge, jnp.int32(1), jnp.int32(0))
        bel = jnp.where(ge, jnp.int32(0), bits)
        cnts.append(c[:32] + c[32:])
        belows.append(jnp.maximum(bel[:32], bel[32:]))
    cnt_t = _colsum(_tree(cnts, lambda a, b: a + b))
    below = _tree(belows, jnp.maximum)
    below = jnp.max(jnp.maximum(below[:16], below[16:]), axis=0, keepdims=True)

    v_hi = lax.bitcast_convert_type(tbits, jnp.float32)
    v_lo = jnp.where(cnt_t >= target + 1, v_hi,
                     lax.bitcast_convert_type(below, jnp.float32))
    thres = jnp.sqrt(v_lo) * lw + jnp.sqrt(v_hi) * hw

    for s in range(nslab):
        sl = pl.ds(s * _SLAB, _SLAB)
        keep = (jnp.sqrt(n2s[s]) >= thres).astype(jnp.float32)
        for r in range(group):
            o_ref[r, sl] = x_ref[r, sl] * keep


@jax.jit
def kernel(x):
    p, q, k1, k2 = x.shape
    group = k1 * k2
    qblk = 128
    grid = q // qblk
    xt = x.transpose(2, 3, 0, 1).reshape(group, p, q)

    qs = np.float32(0.9) * np.float32(p - 1)
    low = np.floor(qs)
    hw = np.float32(qs - low)
    lw = np.float32(np.float32(1.0) - hw)
    target = int(p - np.ceil(qs))

    body = functools.partial(_strip_body, target=target, lw=lw, hw=hw)
    out = pl.pallas_call(
        body,
        grid=(grid,),
        in_specs=[pl.BlockSpec((group, p, qblk), lambda i: (0, 0, i))],
        out_specs=pl.BlockSpec((group, p, qblk), lambda i: (0, 0, i)),
        out_shape=jax.ShapeDtypeStruct((group, p, q), jnp.float32),
    )(xt)
    return out.reshape(k1, k2, p, q).transpose(2, 3, 0, 1)
